# baseline (speedup 1.0000x reference)
import functools

import jax
import jax.numpy as jnp
from jax import lax
from jax.experimental import pallas as pl
from jax.experimental.pallas import tpu as pltpu
from jax.experimental.pallas import tpu_sc as plsc

B = 128
V = 100000
NC = 2
NS = 16
NW = NC * NS
ROWS_PER_W = B // NW
L = 16

CHUNK = 10000
NCHUNK = V // CHUNK

_BIG = 2**30


def _sc_argmax(x_hbm, out_hbm, buf, res):
    cid = lax.axis_index("c")
    sid = lax.axis_index("s")
    wid = sid * NC + cid

    acc = jnp.zeros((L,), jnp.int32)
    for r in range(ROWS_PER_W):
        row = wid * ROWS_PER_W + r

        def chunk_body(ci, carry):
            mx, mi = carry
            pltpu.sync_copy(x_hbm.at[pl.ds(row * V + ci * CHUNK, CHUNK)], buf)
            base = ci * CHUNK

            def body(i, c2):
                m2, i2 = c2
                xv = buf[pl.ds(i * L, L)]
                idx = lax.iota(jnp.int32, L) + (base + i * L)
                upd = xv > m2
                m2 = jnp.where(upd, xv, m2)
                i2 = jnp.where(upd, idx, i2)
                return (m2, i2)

            return lax.fori_loop(0, CHUNK // L, body, (mx, mi))

        init = (jnp.full((L,), -jnp.inf, jnp.float32),
                jnp.zeros((L,), jnp.int32))
        mx, mi = lax.fori_loop(0, NCHUNK, chunk_body, init)

        m = jnp.max(mx)
        cand = jnp.where(mx == m, mi, _BIG)
        acc = jnp.where(lax.iota(jnp.int32, L) == r, jnp.min(cand), acc)

    res[...] = acc
    pltpu.sync_copy(res, out_hbm.at[wid])


@jax.jit
def kernel(x):
    mesh = plsc.VectorSubcoreMesh(core_axis_name="c", subcore_axis_name="s")
    out = pl.kernel(
        _sc_argmax,
        out_type=jax.ShapeDtypeStruct((NW, L), jnp.int32),
        mesh=mesh,
        scratch_types=[
            pltpu.VMEM((CHUNK,), jnp.float32),
            pltpu.VMEM((L,), jnp.int32),
        ],
        compiler_params=pltpu.CompilerParams(needs_layout_passes=False),
    )(x.reshape(B * V))
    return out[:, :ROWS_PER_W].reshape(B).astype(jnp.int64)


# speedup vs baseline: 1.7305x; 1.7305x over previous
import jax
import jax.numpy as jnp
from jax import lax
from jax.experimental import pallas as pl
from jax.experimental.pallas import tpu as pltpu
from jax.experimental.pallas import tpu_sc as plsc

B = 128
V = 100000
NC = 2
NS = 16
NW = NC * NS
ROWS_PER_W = B // NW
L = 16

CHUNK = 20000
NCHUNK = V // CHUNK
NBUF = 6
NVREG = CHUNK // L
G1 = 10
G2 = 5

_BIG = 2**30
_TOTAL = ROWS_PER_W * NCHUNK


def _tree_max(vs):
    while len(vs) > 1:
        vs = [jnp.maximum(vs[i], vs[i + 1]) for i in range(0, len(vs) - 1, 2)] \
             + ([vs[-1]] if len(vs) % 2 else [])
    return vs[0]


def _sc_argmax(x_hbm, out_hbm, *scratch):
    bufs = scratch[:NBUF]
    res = scratch[NBUF]
    minbuf = scratch[NBUF + 1]
    sem = scratch[NBUF + 2]

    cid = lax.axis_index("c")
    sid = lax.axis_index("s")
    wid = sid * NC + cid
    row0 = wid * ROWS_PER_W

    iota16 = lax.iota(jnp.int32, L)
    neginf = jnp.full((L,), -jnp.inf, jnp.float32)

    def copy(j):
        row = j // NCHUNK
        c = j % NCHUNK
        src = x_hbm.at[pl.ds((row0 + row) * V + c * CHUNK, CHUNK)]
        return pltpu.make_async_copy(src, bufs[j % NBUF], sem)

    copy(0).start()

    acc_out = jnp.zeros((L,), jnp.int32)
    mcs = []
    for j in range(_TOTAL):
        r, c = divmod(j, NCHUNK)
        buf = bufs[j % NBUF]
        copy(j).wait()
        if j + 1 < _TOTAL:
            copy(j + 1).start()

        @plsc.parallel_loop(0, NVREG, step=G1,
                            carry=tuple(neginf for _ in range(G1)))
        def _p1(i, accs, buf=buf):
            return tuple(
                jnp.maximum(accs[k], buf[pl.ds((i + k) * L, L)])
                for k in range(G1)
            )

        mcs.append(_tree_max(list(_p1)))

        if c == NCHUNK - 1:
            rowmax = _tree_max(list(mcs))
            m = jnp.max(rowmax)
            hits = [jnp.any(mc == m) for mc in mcs]
            winner = jnp.int32(NCHUNK - 1)
            for cc in range(NCHUNK - 2, -1, -1):
                winner = jnp.where(hits[cc], jnp.int32(cc), winner)

            for cc in range(NCHUNK):
                wbuf = bufs[(r * NCHUNK + cc) % NBUF]

                @pl.when(winner == cc)
                def _scan(wbuf=wbuf, cc=cc):
                    @plsc.parallel_loop(0, NVREG, step=G2,
                                        carry=tuple(
                                            jnp.full((L,), _BIG, jnp.int32)
                                            for _ in range(G2)))
                    def _p2(i, minaccs):
                        outs = []
                        for k in range(G2):
                            xv = wbuf[pl.ds((i + k) * L, L)]
                            idxv = iota16 + (cc * CHUNK + (i + k) * L)
                            cand = jnp.where(xv == m, idxv, _BIG)
                            outs.append(jnp.minimum(minaccs[k], cand))
                        return tuple(outs)

                    minvec = _p2[0]
                    for v in _p2[1:]:
                        minvec = jnp.minimum(minvec, v)
                    minbuf[...] = minvec

            idx_r = jnp.min(minbuf[...])
            acc_out = jnp.where(iota16 == r, idx_r, acc_out)
            mcs = []

    res[...] = acc_out
    pltpu.sync_copy(res, out_hbm.at[wid])


@jax.jit
def kernel(x):
    mesh = plsc.VectorSubcoreMesh(core_axis_name="c", subcore_axis_name="s")
    out = pl.kernel(
        _sc_argmax,
        out_type=jax.ShapeDtypeStruct((NW, L), jnp.int32),
        mesh=mesh,
        scratch_types=[pltpu.VMEM((CHUNK,), jnp.float32) for _ in range(NBUF)]
        + [
            pltpu.VMEM((L,), jnp.int32),
            pltpu.VMEM((L,), jnp.int32),
            pltpu.SemaphoreType.DMA,
        ],
        compiler_params=pltpu.CompilerParams(needs_layout_passes=False),
    )(x.reshape(B * V))
    return out[:, :ROWS_PER_W].reshape(B).astype(jnp.int64)
